# baseline (speedup 1.0000x reference)
import functools

import jax
import jax.numpy as jnp
from jax import lax
from jax.experimental import pallas as pl
from jax.experimental.pallas import tpu as pltpu
from jax.experimental.pallas import tpu_sc as plsc

N_SAMPLE = 2048
KNN = 4
B, D, N = 4, 128, 8192

_R = 32
_L = N * B // _R


def _combine8(v, op):
    v = op(v[0:16, :], v[16:32, :])
    v = op(v[0:8, :], v[8:16, :])
    return op(v[0:4, :], v[4:8, :])


def _bcast8(v4):
    return jnp.concatenate([v4] * 8, axis=0)


def _fps_body(xyz_ref, ind_ref, sxyz_ref, dist_ref):
    px = xyz_ref[0]
    py = xyz_ref[1]
    pz = xyz_ref[2]
    row = lax.broadcasted_iota(jnp.int32, (_R, _L), 0)
    lane = lax.broadcasted_iota(jnp.int32, (_R, _L), 1)
    iota_n = (row // B) * _L + lane
    dist_ref[...] = jnp.full((_R, _L), jnp.inf, dtype=jnp.float32)

    def step(i, far_col):
        sel = iota_n == far_col
        cx = _combine8(jnp.sum(jnp.where(sel, px, 0.0), 1, keepdims=True),
                       jnp.add)
        cy = _combine8(jnp.sum(jnp.where(sel, py, 0.0), 1, keepdims=True),
                       jnp.add)
        cz = _combine8(jnp.sum(jnp.where(sel, pz, 0.0), 1, keepdims=True),
                       jnp.add)
        sxyz_ref[pl.ds(i, 1), :, :] = jnp.concatenate(
            [cx, cy, cz], axis=1).T.reshape(1, 3, B)
        dx = px - _bcast8(cx)
        dy = py - _bcast8(cy)
        dz = pz - _bcast8(cz)
        d = (dx * dx + dy * dy) + dz * dz
        dist = jnp.minimum(dist_ref[...], d)
        dist_ref[...] = dist
        mx = _bcast8(_combine8(jnp.max(dist, 1, keepdims=True), jnp.maximum))
        arg = _combine8(
            jnp.min(jnp.where(dist == mx, iota_n, N), 1, keepdims=True),
            jnp.minimum).astype(jnp.int32)
        ind_ref[pl.ds(i + 1, 1), :] = arg.reshape(1, B)
        return _bcast8(arg)

    far0 = jnp.zeros((_R, 1), jnp.int32)
    ind_ref[pl.ds(0, 1), :] = jnp.zeros((1, B), jnp.int32)
    lax.fori_loop(0, N_SAMPLE, step, far0, unroll=2)


def _run_fps(xyz_r):
    return pl.pallas_call(
        _fps_body,
        out_shape=(
            jax.ShapeDtypeStruct((N_SAMPLE + 1, B), jnp.int32),
            jax.ShapeDtypeStruct((N_SAMPLE, 3, B), jnp.float32),
        ),
        scratch_shapes=[pltpu.VMEM((_R, _L), jnp.float32)],
    )(xyz_r)


_BM = 256
_NCH = 2048


def _bf16_rne(v):
    u = lax.bitcast_convert_type(v, jnp.uint32)
    lsb = (u >> 16) & jnp.uint32(1)
    r = (u + jnp.uint32(0x7FFF) + lsb) & jnp.uint32(0xFFFF0000)
    return lax.bitcast_convert_type(r, jnp.float32)


def _knn_body(xyz_ref, sxyz_ref, nbr_ref):
    sx = sxyz_ref[0, 0, :].reshape(_BM, 1)
    sy = sxyz_ref[0, 1, :].reshape(_BM, 1)
    sz = sxyz_ref[0, 2, :].reshape(_BM, 1)
    a2 = (sx * sx + sy * sy) + sz * sz
    sxr = _bf16_rne(sx)
    syr = _bf16_rne(sy)
    szr = _bf16_rne(sz)

    big = jnp.float32(jnp.inf)
    best_v = [jnp.full((_BM, 1), big, jnp.float32) for _ in range(KNN)]
    best_i = [jnp.full((_BM, 1), N, jnp.int32) for _ in range(KNN)]

    for c in range(N // _NCH):
        n0 = c * _NCH
        px = xyz_ref[0, 0, pl.ds(n0, _NCH)].reshape(1, _NCH)
        py = xyz_ref[0, 1, pl.ds(n0, _NCH)].reshape(1, _NCH)
        pz = xyz_ref[0, 2, pl.ds(n0, _NCH)].reshape(1, _NCH)
        b2 = (px * px + py * py) + pz * pz
        pxr = _bf16_rne(px)
        pyr = _bf16_rne(py)
        pzr = _bf16_rne(pz)
        ab = (pxr * sxr + pyr * syr) + pzr * szr
        d2 = jnp.sqrt(jnp.maximum((a2 + b2) - 2.0 * ab, 0.0))
        iota = lax.broadcasted_iota(jnp.int32, (_BM, _NCH), 1) + n0
        for _ in range(KNN):
            mn = jnp.min(d2, axis=1, keepdims=True)
            arg = jnp.min(
                jnp.where(d2 == mn, iota, N), axis=1, keepdims=True)
            cv, ci = mn, arg
            for k in range(KNN):
                take = cv < best_v[k]
                nv = jnp.where(take, cv, best_v[k])
                ni = jnp.where(take, ci, best_i[k])
                cv = jnp.where(take, best_v[k], cv)
                ci = jnp.where(take, best_i[k], ci)
                best_v[k], best_i[k] = nv, ni
            d2 = jnp.where(iota == arg, big, d2)

    for k in range(KNN):
        nbr_ref[0, :, k] = best_i[k][:, 0]


def _run_knn(xyz, sxyz0):
    grid = (B, N_SAMPLE // _BM)
    return pl.pallas_call(
        _knn_body,
        grid=grid,
        in_specs=[
            pl.BlockSpec((1, 3, N), lambda b, m: (b, 0, 0)),
            pl.BlockSpec((1, 3, _BM), lambda b, m: (b, 0, m)),
        ],
        out_specs=pl.BlockSpec((1, _BM, KNN), lambda b, m: (b, m, 0)),
        out_shape=jax.ShapeDtypeStruct((B, N_SAMPLE, KNN), jnp.int32),
    )(xyz, sxyz0)


_XW = 16


def _gather_body(xt_ref, xyzp_ref, nbr_ref, sx_ref, sxyz_ref,
                 idx_v, rows_v, xrows_v, out_v, oxyz_v, sem1, sem2):
    info = plsc.get_sparse_core_info()
    nw = info.num_cores * info.num_subcores
    wid = lax.axis_index("s") * info.num_cores + lax.axis_index("c")
    rows_total = B * N_SAMPLE
    rows_per_w = rows_total // nw
    m_chunk = 32
    n_chunks = rows_per_w // m_chunk
    base_row = wid * rows_per_w
    batch = base_row // N_SAMPLE
    n_off = batch * N

    def do_chunk(ci, _):
        row0 = base_row + ci * m_chunk
        pltpu.sync_copy(nbr_ref.at[pl.ds(row0 * KNN, m_chunk * KNN)], idx_v)
        for j in range(m_chunk * KNN // 16):
            sl = pl.ds(j * 16, 16)
            idx_v[sl] = idx_v[sl] + n_off
        pltpu.async_copy(xt_ref.at[idx_v], rows_v, sem1).wait()
        pltpu.async_copy(xyzp_ref.at[idx_v], xrows_v, sem2).wait()
        quarter = jnp.float32(0.25)
        for m in range(m_chunk):
            r = m * KNN
            for j in range(D // 16):
                sl = pl.ds(j * 16, 16)
                v = jnp.maximum(
                    jnp.maximum(rows_v[r, sl], rows_v[r + 1, sl]),
                    jnp.maximum(rows_v[r + 2, sl], rows_v[r + 3, sl]))
                out_v[m, sl] = v
            sl0 = pl.ds(0, _XW)
            s = ((xrows_v[r, sl0] + xrows_v[r + 1, sl0])
                 + (xrows_v[r + 2, sl0] + xrows_v[r + 3, sl0]))
            oxyz_v[m, :] = s * quarter
        pltpu.sync_copy(out_v, sx_ref.at[pl.ds(row0, m_chunk)])
        pltpu.sync_copy(oxyz_v, sxyz_ref.at[pl.ds(row0, m_chunk)])
        return ()

    lax.fori_loop(0, n_chunks, do_chunk, ())


def _run_gather(xt, xyzp, nbr_flat):
    mesh = plsc.VectorSubcoreMesh(core_axis_name="c", subcore_axis_name="s")
    kfn = pl.kernel(
        _gather_body,
        mesh=mesh,
        out_type=(
            jax.ShapeDtypeStruct((B * N_SAMPLE, D), jnp.float32),
            jax.ShapeDtypeStruct((B * N_SAMPLE, _XW), jnp.float32),
        ),
        scratch_types=[
            pltpu.VMEM((32 * KNN,), jnp.int32),
            pltpu.VMEM((32 * KNN, D), jnp.float32),
            pltpu.VMEM((32 * KNN, D), jnp.float32),
            pltpu.VMEM((32, D), jnp.float32),
            pltpu.VMEM((32, _XW), jnp.float32),
            pltpu.SemaphoreType.DMA,
            pltpu.SemaphoreType.DMA,
        ],
    )
    return kfn(xt, xyzp, nbr_flat)


@jax.jit
def kernel(x, xyz):
    xyz_r = jnp.transpose(
        xyz.reshape(B, 3, _R // B, _L), (1, 2, 0, 3)).reshape(3, _R, _L)
    ind_t, sxyz_t = _run_fps(xyz_r)
    sample_ind = ind_t[:N_SAMPLE].T
    sxyz0 = jnp.transpose(sxyz_t, (2, 1, 0))
    neighbor_ind = _run_knn(xyz, sxyz0)

    xyz_t = jnp.transpose(xyz, (0, 2, 1))
    xt = jnp.transpose(x, (0, 2, 1)).reshape(B * N, D)
    xyzp = jnp.concatenate(
        [xyz_t, jnp.zeros((B, N, D - 3), jnp.float32)], axis=2
    ).reshape(B * N, D)
    nbr_flat = neighbor_ind.reshape(B * N_SAMPLE * KNN)
    sx_rows, sxyz_rows = _run_gather(xt, xyzp, nbr_flat)
    sample_x = jnp.transpose(
        sx_rows.reshape(B, N_SAMPLE, D), (0, 2, 1))
    sample_xyz = jnp.transpose(
        sxyz_rows.reshape(B, N_SAMPLE, _XW)[:, :, :3], (0, 2, 1))
    return (sample_x, sample_xyz, sample_ind, neighbor_ind)


# speedup vs baseline: 1.1033x; 1.1033x over previous
import functools

import jax
import jax.numpy as jnp
from jax import lax
from jax.experimental import pallas as pl
from jax.experimental.pallas import tpu as pltpu
from jax.experimental.pallas import tpu_sc as plsc

N_SAMPLE = 2048
KNN = 4
B, D, N = 4, 128, 8192

_R = 8
_L = N * B // _R


def _fps_body(xyz_ref, ind_ref, sxyz_ref, dist_ref):
    px = xyz_ref[0]
    py = xyz_ref[1]
    pz = xyz_ref[2]
    row = lax.broadcasted_iota(jnp.int32, (_R, _L), 0)
    lane = lax.broadcasted_iota(jnp.int32, (_R, _L), 1)
    iota_n = (row // B) * _L + lane
    dist_ref[...] = jnp.full((_R, _L), jnp.inf, dtype=jnp.float32)

    def step(i, far_col):
        sel = iota_n == far_col
        cx = jnp.sum(jnp.where(sel, px, 0.0), 1, keepdims=True)
        cy = jnp.sum(jnp.where(sel, py, 0.0), 1, keepdims=True)
        cz = jnp.sum(jnp.where(sel, pz, 0.0), 1, keepdims=True)
        cx = cx[0:B, :] + cx[B:_R, :]
        cy = cy[0:B, :] + cy[B:_R, :]
        cz = cz[0:B, :] + cz[B:_R, :]
        sxyz_ref[pl.ds(i, 1), :, pl.ds(0, 1)] = cx.reshape(1, B, 1)
        sxyz_ref[pl.ds(i, 1), :, pl.ds(1, 1)] = cy.reshape(1, B, 1)
        sxyz_ref[pl.ds(i, 1), :, pl.ds(2, 1)] = cz.reshape(1, B, 1)
        cfx = jnp.concatenate([cx, cx], axis=0)
        cfy = jnp.concatenate([cy, cy], axis=0)
        cfz = jnp.concatenate([cz, cz], axis=0)
        dx = px - cfx
        dy = py - cfy
        dz = pz - cfz
        d = (dx * dx + dy * dy) + dz * dz
        dist = jnp.minimum(dist_ref[...], d)
        dist_ref[...] = dist
        rmax = jnp.max(dist, 1, keepdims=True)
        rarg = jnp.min(jnp.where(dist == rmax, iota_n, N), 1,
                       keepdims=True)
        va, ia = rmax[0:B, :], rarg[0:B, :]
        vb, ib = rmax[B:_R, :], rarg[B:_R, :]
        bwins = (vb > va) | ((vb == va) & (ib < ia))
        arg = jnp.where(bwins, ib, ia).astype(jnp.int32)
        ind_ref[pl.ds(i + 1, 1), :, :] = arg.reshape(1, B, 1)
        return jnp.concatenate([arg, arg], axis=0)

    far0 = jnp.zeros((_R, 1), jnp.int32)
    ind_ref[pl.ds(0, 1), :, :] = jnp.zeros((1, B, 1), jnp.int32)
    lax.fori_loop(0, N_SAMPLE, step, far0)


def _run_fps(xyz_r):
    return pl.pallas_call(
        _fps_body,
        out_shape=(
            jax.ShapeDtypeStruct((N_SAMPLE + 1, B, 1), jnp.int32),
            jax.ShapeDtypeStruct((N_SAMPLE, B, 3), jnp.float32),
        ),
        scratch_shapes=[pltpu.VMEM((_R, _L), jnp.float32)],
    )(xyz_r)


_BM = 256
_NCH = 2048


def _bf16_rne(v):
    u = lax.bitcast_convert_type(v, jnp.uint32)
    lsb = (u >> 16) & jnp.uint32(1)
    r = (u + jnp.uint32(0x7FFF) + lsb) & jnp.uint32(0xFFFF0000)
    return lax.bitcast_convert_type(r, jnp.float32)


def _knn_body(xyz_ref, sxyz_ref, nbr_ref):
    sx = sxyz_ref[0, 0, :].reshape(_BM, 1)
    sy = sxyz_ref[0, 1, :].reshape(_BM, 1)
    sz = sxyz_ref[0, 2, :].reshape(_BM, 1)
    a2 = (sx * sx + sy * sy) + sz * sz
    sxr = _bf16_rne(sx)
    syr = _bf16_rne(sy)
    szr = _bf16_rne(sz)

    big = jnp.float32(jnp.inf)
    best_v = [jnp.full((_BM, 1), big, jnp.float32) for _ in range(KNN)]
    best_i = [jnp.full((_BM, 1), N, jnp.int32) for _ in range(KNN)]

    for c in range(N // _NCH):
        n0 = c * _NCH
        px = xyz_ref[0, 0, pl.ds(n0, _NCH)].reshape(1, _NCH)
        py = xyz_ref[0, 1, pl.ds(n0, _NCH)].reshape(1, _NCH)
        pz = xyz_ref[0, 2, pl.ds(n0, _NCH)].reshape(1, _NCH)
        b2 = (px * px + py * py) + pz * pz
        pxr = _bf16_rne(px)
        pyr = _bf16_rne(py)
        pzr = _bf16_rne(pz)
        ab = (pxr * sxr + pyr * syr) + pzr * szr
        d2 = jnp.sqrt(jnp.maximum((a2 + b2) - 2.0 * ab, 0.0))
        iota = lax.broadcasted_iota(jnp.int32, (_BM, _NCH), 1) + n0
        for _ in range(KNN):
            mn = jnp.min(d2, axis=1, keepdims=True)
            arg = jnp.min(
                jnp.where(d2 == mn, iota, N), axis=1, keepdims=True)
            cv, ci = mn, arg
            for k in range(KNN):
                take = cv < best_v[k]
                nv = jnp.where(take, cv, best_v[k])
                ni = jnp.where(take, ci, best_i[k])
                cv = jnp.where(take, best_v[k], cv)
                ci = jnp.where(take, best_i[k], ci)
                best_v[k], best_i[k] = nv, ni
            d2 = jnp.where(iota == arg, big, d2)

    for k in range(KNN):
        nbr_ref[0, :, k] = best_i[k][:, 0]


def _run_knn(xyz, sxyz0):
    grid = (B, N_SAMPLE // _BM)
    return pl.pallas_call(
        _knn_body,
        grid=grid,
        in_specs=[
            pl.BlockSpec((1, 3, N), lambda b, m: (b, 0, 0)),
            pl.BlockSpec((1, 3, _BM), lambda b, m: (b, 0, m)),
        ],
        out_specs=pl.BlockSpec((1, _BM, KNN), lambda b, m: (b, m, 0)),
        out_shape=jax.ShapeDtypeStruct((B, N_SAMPLE, KNN), jnp.int32),
    )(xyz, sxyz0)


_XW = 16


def _gather_body(xt_ref, xyzp_ref, nbr_ref, sx_ref, sxyz_ref,
                 idx_v, rows_v, xrows_v, out_v, oxyz_v, sem1, sem2):
    info = plsc.get_sparse_core_info()
    nw = info.num_cores * info.num_subcores
    wid = lax.axis_index("s") * info.num_cores + lax.axis_index("c")
    rows_total = B * N_SAMPLE
    rows_per_w = rows_total // nw
    m_chunk = 32
    n_chunks = rows_per_w // m_chunk
    base_row = wid * rows_per_w
    batch = base_row // N_SAMPLE
    n_off = batch * N

    def do_chunk(ci, _):
        row0 = base_row + ci * m_chunk
        pltpu.sync_copy(nbr_ref.at[pl.ds(row0 * KNN, m_chunk * KNN)], idx_v)
        for j in range(m_chunk * KNN // 16):
            sl = pl.ds(j * 16, 16)
            idx_v[sl] = idx_v[sl] + n_off
        pltpu.async_copy(xt_ref.at[idx_v], rows_v, sem1).wait()
        pltpu.async_copy(xyzp_ref.at[idx_v], xrows_v, sem2).wait()
        quarter = jnp.float32(0.25)
        for m in range(m_chunk):
            r = m * KNN
            for j in range(D // 16):
                sl = pl.ds(j * 16, 16)
                v = jnp.maximum(
                    jnp.maximum(rows_v[r, sl], rows_v[r + 1, sl]),
                    jnp.maximum(rows_v[r + 2, sl], rows_v[r + 3, sl]))
                out_v[m, sl] = v
            sl0 = pl.ds(0, _XW)
            s = ((xrows_v[r, sl0] + xrows_v[r + 1, sl0])
                 + (xrows_v[r + 2, sl0] + xrows_v[r + 3, sl0]))
            oxyz_v[m, :] = s * quarter
        pltpu.sync_copy(out_v, sx_ref.at[pl.ds(row0, m_chunk)])
        pltpu.sync_copy(oxyz_v, sxyz_ref.at[pl.ds(row0, m_chunk)])
        return ()

    lax.fori_loop(0, n_chunks, do_chunk, ())


def _run_gather(xt, xyzp, nbr_flat):
    mesh = plsc.VectorSubcoreMesh(core_axis_name="c", subcore_axis_name="s")
    kfn = pl.kernel(
        _gather_body,
        mesh=mesh,
        out_type=(
            jax.ShapeDtypeStruct((B * N_SAMPLE, D), jnp.float32),
            jax.ShapeDtypeStruct((B * N_SAMPLE, _XW), jnp.float32),
        ),
        scratch_types=[
            pltpu.VMEM((32 * KNN,), jnp.int32),
            pltpu.VMEM((32 * KNN, D), jnp.float32),
            pltpu.VMEM((32 * KNN, D), jnp.float32),
            pltpu.VMEM((32, D), jnp.float32),
            pltpu.VMEM((32, _XW), jnp.float32),
            pltpu.SemaphoreType.DMA,
            pltpu.SemaphoreType.DMA,
        ],
    )
    return kfn(xt, xyzp, nbr_flat)


@jax.jit
def kernel(x, xyz):
    xyz_r = jnp.transpose(
        xyz.reshape(B, 3, _R // B, _L), (1, 2, 0, 3)).reshape(3, _R, _L)
    ind_t, sxyz_t = _run_fps(xyz_r)
    sample_ind = ind_t[:N_SAMPLE, :, 0].T
    sxyz0 = jnp.transpose(sxyz_t, (1, 2, 0))
    neighbor_ind = _run_knn(xyz, sxyz0)

    xyz_t = jnp.transpose(xyz, (0, 2, 1))
    xt = jnp.transpose(x, (0, 2, 1)).reshape(B * N, D)
    xyzp = jnp.concatenate(
        [xyz_t, jnp.zeros((B, N, D - 3), jnp.float32)], axis=2
    ).reshape(B * N, D)
    nbr_flat = neighbor_ind.reshape(B * N_SAMPLE * KNN)
    sx_rows, sxyz_rows = _run_gather(xt, xyzp, nbr_flat)
    sample_x = jnp.transpose(
        sx_rows.reshape(B, N_SAMPLE, D), (0, 2, 1))
    sample_xyz = jnp.transpose(
        sxyz_rows.reshape(B, N_SAMPLE, _XW)[:, :, :3], (0, 2, 1))
    return (sample_x, sample_xyz, sample_ind, neighbor_ind)


# speedup vs baseline: 1.1042x; 1.0008x over previous
import functools

import jax
import jax.numpy as jnp
from jax import lax
from jax.experimental import pallas as pl
from jax.experimental.pallas import tpu as pltpu
from jax.experimental.pallas import tpu_sc as plsc

N_SAMPLE = 2048
KNN = 4
B, D, N = 4, 128, 8192

_R = 8
_L = N * B // _R


def _fps_body(xyz_ref, ind_ref, sxyz_ref, dist_ref):
    px = xyz_ref[0]
    py = xyz_ref[1]
    pz = xyz_ref[2]
    row = lax.broadcasted_iota(jnp.int32, (_R, _L), 0)
    lane = lax.broadcasted_iota(jnp.int32, (_R, _L), 1)
    iota_n = (row // B) * _L + lane
    dist_ref[...] = jnp.full((_R, _L), jnp.inf, dtype=jnp.float32)

    def step(i, far_col):
        sel = iota_n == far_col
        cx = jnp.sum(jnp.where(sel, px, 0.0), 1, keepdims=True)
        cy = jnp.sum(jnp.where(sel, py, 0.0), 1, keepdims=True)
        cz = jnp.sum(jnp.where(sel, pz, 0.0), 1, keepdims=True)
        cx = cx[0:B, :] + cx[B:_R, :]
        cy = cy[0:B, :] + cy[B:_R, :]
        cz = cz[0:B, :] + cz[B:_R, :]
        sxyz_ref[pl.ds(i, 1), :, pl.ds(0, 1)] = cx.reshape(1, B, 1)
        sxyz_ref[pl.ds(i, 1), :, pl.ds(1, 1)] = cy.reshape(1, B, 1)
        sxyz_ref[pl.ds(i, 1), :, pl.ds(2, 1)] = cz.reshape(1, B, 1)
        cfx = jnp.concatenate([cx, cx], axis=0)
        cfy = jnp.concatenate([cy, cy], axis=0)
        cfz = jnp.concatenate([cz, cz], axis=0)
        dx = px - cfx
        dy = py - cfy
        dz = pz - cfz
        d = (dx * dx + dy * dy) + dz * dz
        dist = jnp.minimum(dist_ref[...], d)
        dist_ref[...] = dist
        rmax = jnp.max(dist, 1, keepdims=True)
        rarg = jnp.min(jnp.where(dist == rmax, iota_n, N), 1,
                       keepdims=True)
        va, ia = rmax[0:B, :], rarg[0:B, :]
        vb, ib = rmax[B:_R, :], rarg[B:_R, :]
        bwins = (vb > va) | ((vb == va) & (ib < ia))
        arg = jnp.where(bwins, ib, ia).astype(jnp.int32)
        ind_ref[pl.ds(i + 1, 1), :, :] = arg.reshape(1, B, 1)
        return jnp.concatenate([arg, arg], axis=0)

    far0 = jnp.zeros((_R, 1), jnp.int32)
    ind_ref[pl.ds(0, 1), :, :] = jnp.zeros((1, B, 1), jnp.int32)
    lax.fori_loop(0, N_SAMPLE, step, far0)


def _run_fps(xyz_r):
    return pl.pallas_call(
        _fps_body,
        out_shape=(
            jax.ShapeDtypeStruct((N_SAMPLE + 1, B, 1), jnp.int32),
            jax.ShapeDtypeStruct((N_SAMPLE, B, 3), jnp.float32),
        ),
        scratch_shapes=[pltpu.VMEM((_R, _L), jnp.float32)],
    )(xyz_r)


_BM = 256
_NCH = 2048


def _bf16_rne(v):
    u = lax.bitcast_convert_type(v, jnp.uint32)
    lsb = (u >> 16) & jnp.uint32(1)
    r = (u + jnp.uint32(0x7FFF) + lsb) & jnp.uint32(0xFFFF0000)
    return lax.bitcast_convert_type(r, jnp.float32)


def _knn_body(xyz_ref, sxyz_ref, nbr_ref):
    sx = sxyz_ref[0, 0, :].reshape(_BM, 1)
    sy = sxyz_ref[0, 1, :].reshape(_BM, 1)
    sz = sxyz_ref[0, 2, :].reshape(_BM, 1)
    a2 = (sx * sx + sy * sy) + sz * sz
    sxr = _bf16_rne(sx)
    syr = _bf16_rne(sy)
    szr = _bf16_rne(sz)

    big = jnp.float32(jnp.inf)
    best_v = [jnp.full((_BM, 1), big, jnp.float32) for _ in range(KNN)]
    best_i = [jnp.full((_BM, 1), N, jnp.int32) for _ in range(KNN)]

    for c in range(N // _NCH):
        n0 = c * _NCH
        px = xyz_ref[0, 0, pl.ds(n0, _NCH)].reshape(1, _NCH)
        py = xyz_ref[0, 1, pl.ds(n0, _NCH)].reshape(1, _NCH)
        pz = xyz_ref[0, 2, pl.ds(n0, _NCH)].reshape(1, _NCH)
        b2 = (px * px + py * py) + pz * pz
        pxr = _bf16_rne(px)
        pyr = _bf16_rne(py)
        pzr = _bf16_rne(pz)
        ab = (pxr * sxr + pyr * syr) + pzr * szr
        d2 = jnp.sqrt(jnp.maximum((a2 + b2) - 2.0 * ab, 0.0))
        iota = lax.broadcasted_iota(jnp.int32, (_BM, _NCH), 1) + n0
        for p in range(KNN):
            mn = jnp.min(d2, axis=1, keepdims=True)
            arg = jnp.min(
                jnp.where(d2 == mn, iota, N), axis=1, keepdims=True)
            cv, ci = mn, arg
            for k in range(KNN):
                take = cv < best_v[k]
                nv = jnp.where(take, cv, best_v[k])
                ni = jnp.where(take, ci, best_i[k])
                cv = jnp.where(take, best_v[k], cv)
                ci = jnp.where(take, best_i[k], ci)
                best_v[k], best_i[k] = nv, ni
            if p + 1 < KNN:
                d2 = jnp.where(iota == arg, big, d2)

    for k in range(KNN):
        nbr_ref[0, :, k] = best_i[k][:, 0]


def _run_knn(xyz, sxyz0):
    grid = (B, N_SAMPLE // _BM)
    return pl.pallas_call(
        _knn_body,
        grid=grid,
        in_specs=[
            pl.BlockSpec((1, 3, N), lambda b, m: (b, 0, 0)),
            pl.BlockSpec((1, 3, _BM), lambda b, m: (b, 0, m)),
        ],
        out_specs=pl.BlockSpec((1, _BM, KNN), lambda b, m: (b, m, 0)),
        out_shape=jax.ShapeDtypeStruct((B, N_SAMPLE, KNN), jnp.int32),
    )(xyz, sxyz0)


_XW = 16


def _gather_body(xt_ref, xyzp_ref, nbr_ref, sx_ref, sxyz_ref,
                 idx_v, rows_v, xrows_v, out_v, oxyz_v, sem1, sem2):
    info = plsc.get_sparse_core_info()
    nw = info.num_cores * info.num_subcores
    wid = lax.axis_index("s") * info.num_cores + lax.axis_index("c")
    rows_total = B * N_SAMPLE
    rows_per_w = rows_total // nw
    m_chunk = 32
    n_chunks = rows_per_w // m_chunk
    base_row = wid * rows_per_w
    batch = base_row // N_SAMPLE
    n_off = batch * N

    def do_chunk(ci, _):
        row0 = base_row + ci * m_chunk
        pltpu.sync_copy(nbr_ref.at[pl.ds(row0 * KNN, m_chunk * KNN)], idx_v)
        for j in range(m_chunk * KNN // 16):
            sl = pl.ds(j * 16, 16)
            idx_v[sl] = idx_v[sl] + n_off
        pltpu.async_copy(xt_ref.at[idx_v], rows_v, sem1).wait()
        pltpu.async_copy(xyzp_ref.at[idx_v], xrows_v, sem2).wait()
        quarter = jnp.float32(0.25)
        for m in range(m_chunk):
            r = m * KNN
            for j in range(D // 16):
                sl = pl.ds(j * 16, 16)
                v = jnp.maximum(
                    jnp.maximum(rows_v[r, sl], rows_v[r + 1, sl]),
                    jnp.maximum(rows_v[r + 2, sl], rows_v[r + 3, sl]))
                out_v[m, sl] = v
            sl0 = pl.ds(0, _XW)
            s = ((xrows_v[r, sl0] + xrows_v[r + 1, sl0])
                 + (xrows_v[r + 2, sl0] + xrows_v[r + 3, sl0]))
            oxyz_v[m, :] = s * quarter
        pltpu.sync_copy(out_v, sx_ref.at[pl.ds(row0, m_chunk)])
        pltpu.sync_copy(oxyz_v, sxyz_ref.at[pl.ds(row0, m_chunk)])
        return ()

    lax.fori_loop(0, n_chunks, do_chunk, ())


def _run_gather(xt, xyzp, nbr_flat):
    mesh = plsc.VectorSubcoreMesh(core_axis_name="c", subcore_axis_name="s")
    kfn = pl.kernel(
        _gather_body,
        mesh=mesh,
        out_type=(
            jax.ShapeDtypeStruct((B * N_SAMPLE, D), jnp.float32),
            jax.ShapeDtypeStruct((B * N_SAMPLE, _XW), jnp.float32),
        ),
        scratch_types=[
            pltpu.VMEM((32 * KNN,), jnp.int32),
            pltpu.VMEM((32 * KNN, D), jnp.float32),
            pltpu.VMEM((32 * KNN, D), jnp.float32),
            pltpu.VMEM((32, D), jnp.float32),
            pltpu.VMEM((32, _XW), jnp.float32),
            pltpu.SemaphoreType.DMA,
            pltpu.SemaphoreType.DMA,
        ],
    )
    return kfn(xt, xyzp, nbr_flat)


@jax.jit
def kernel(x, xyz):
    xyz_r = jnp.transpose(
        xyz.reshape(B, 3, _R // B, _L), (1, 2, 0, 3)).reshape(3, _R, _L)
    ind_t, sxyz_t = _run_fps(xyz_r)
    sample_ind = ind_t[:N_SAMPLE, :, 0].T
    sxyz0 = jnp.transpose(sxyz_t, (1, 2, 0))
    neighbor_ind = _run_knn(xyz, sxyz0)

    xyz_t = jnp.transpose(xyz, (0, 2, 1))
    xt = jnp.transpose(x, (0, 2, 1)).reshape(B * N, D)
    xyzp = jnp.concatenate(
        [xyz_t, jnp.zeros((B, N, D - 3), jnp.float32)], axis=2
    ).reshape(B * N, D)
    nbr_flat = neighbor_ind.reshape(B * N_SAMPLE * KNN)
    sx_rows, sxyz_rows = _run_gather(xt, xyzp, nbr_flat)
    sample_x = jnp.transpose(
        sx_rows.reshape(B, N_SAMPLE, D), (0, 2, 1))
    sample_xyz = jnp.transpose(
        sxyz_rows.reshape(B, N_SAMPLE, _XW)[:, :, :3], (0, 2, 1))
    return (sample_x, sample_xyz, sample_ind, neighbor_ind)
